# baseline (speedup 1.0000x reference)
import functools

import jax
import jax.numpy as jnp
from jax import lax
from jax.experimental import pallas as pl
from jax.experimental.pallas import tpu as pltpu
from jax.experimental.pallas import tpu_sc as plsc

_REG_WEIGHT = 1e-4

_NC = 2
_NS = 16
_NW = _NC * _NS
_CHUNK = 512


def _sc_gather(table2, row_idx):
    B = row_idx.shape[0]
    D = table2.shape[1]
    chunks_per_w = B // (_NW * _CHUNK)
    mesh = plsc.VectorSubcoreMesh(core_axis_name="c", subcore_axis_name="s")

    @functools.partial(
        pl.kernel,
        mesh=mesh,
        out_type=jax.ShapeDtypeStruct((B, D), jnp.float32),
        scratch_types=[
            pltpu.VMEM((_CHUNK,), jnp.int32),
            pltpu.VMEM((_CHUNK, D), jnp.float32),
            pltpu.SemaphoreType.DMA,
        ],
    )
    def gather_kernel(table_hbm, idx_hbm, out_hbm, idx_v, rows_v, sem):
        wid = lax.axis_index("s") * _NC + lax.axis_index("c")

        @pl.loop(0, chunks_per_w)
        def _(c):
            base = (wid * chunks_per_w + c) * _CHUNK
            pltpu.sync_copy(idx_hbm.at[pl.ds(base, _CHUNK)], idx_v)
            pltpu.async_copy(table_hbm.at[idx_v], rows_v, sem).wait()
            pltpu.sync_copy(rows_v, out_hbm.at[pl.ds(base, _CHUNK)])

    return gather_kernel(table2, row_idx)


def _loss_body(inv_batch, u_ref, gp_ref, gn_ref, sp_ref, sn_ref, o_ref):
    i = pl.program_id(0)
    u = u_ref[...]
    gp = gp_ref[...]
    gn = gn_ref[...]
    sp = sp_ref[...]
    sn = sn_ref[...]
    d = u.shape[1]
    p = gp[:, :d] + (gp[:, d:] - gp[:, :d]) * sp
    n = gn[:, :d] + (gn[:, d:] - gn[:, :d]) * sn
    diff = jnp.sum(u * (p - n), axis=1)
    loss_terms = -jnp.log(jax.nn.sigmoid(diff))
    l2 = jnp.sum(u * u + p * p + n * n, axis=1)
    part = jnp.sum(loss_terms + _REG_WEIGHT * l2) * inv_batch

    @pl.when(i == 0)
    def _():
        o_ref[...] = jnp.zeros((1, 1), jnp.float32)

    o_ref[...] += jnp.reshape(part, (1, 1))


def kernel(user_embeddings, item_embeddings, pos_items, neg_items):
    batch, d = user_embeddings.shape
    num_items = item_embeddings.shape[0]
    idx = jnp.concatenate(
        [pos_items.astype(jnp.int32), neg_items.astype(jnp.int32)]
    )
    row_idx = lax.shift_right_logical(idx, 1)
    sel = (idx & 1).astype(jnp.float32).reshape(-1, 1)
    table2 = item_embeddings.reshape(num_items // 2, 2 * d)
    rows = _sc_gather(table2, row_idx)

    blk = 2048
    grid = batch // blk
    out = pl.pallas_call(
        functools.partial(_loss_body, 1.0 / batch),
        grid=(grid,),
        in_specs=[
            pl.BlockSpec((blk, d), lambda i: (i, 0)),
            pl.BlockSpec((blk, 2 * d), lambda i: (i, 0)),
            pl.BlockSpec((blk, 2 * d), lambda i, g=grid: (i + g, 0)),
            pl.BlockSpec((blk, 1), lambda i: (i, 0)),
            pl.BlockSpec((blk, 1), lambda i, g=grid: (i + g, 0)),
        ],
        out_specs=pl.BlockSpec((1, 1), lambda i: (0, 0)),
        out_shape=jax.ShapeDtypeStruct((1, 1), jnp.float32),
    )(user_embeddings, rows, rows, sel, sel)
    return out[0, 0]


# speedup vs baseline: 1.7483x; 1.7483x over previous
import functools

import jax
import jax.numpy as jnp
from jax import lax
from jax.experimental import pallas as pl
from jax.experimental.pallas import tpu as pltpu
from jax.experimental.pallas import tpu_sc as plsc

_REG_WEIGHT = 1e-4

_NC = 2
_NS = 16
_NW = _NC * _NS
_CHUNK = 256


def _sc_gather(table, idx):
    B = idx.shape[0]
    D = table.shape[1]
    per_w = B // _NW
    n_chunks = per_w // _CHUNK
    mesh = plsc.VectorSubcoreMesh(core_axis_name="c", subcore_axis_name="s")

    @functools.partial(
        pl.kernel,
        mesh=mesh,
        out_type=jax.ShapeDtypeStruct((B, D), jnp.float32),
        scratch_types=[
            pltpu.VMEM((per_w,), jnp.int32),
            pltpu.VMEM((_CHUNK, D), jnp.float32),
            pltpu.SemaphoreType.DMA,
        ],
    )
    def gather_kernel(table_hbm, idx_hbm, out_hbm, idx_v, rows_v, sem):
        wid = lax.axis_index("s") * _NC + lax.axis_index("c")
        wbase = wid * per_w
        pltpu.sync_copy(idx_hbm.at[pl.ds(wbase, per_w)], idx_v)

        @pl.loop(0, n_chunks)
        def _(c):
            cbase = c * _CHUNK

            @pl.loop(0, _CHUNK, step=16)
            def _(i0):
                v = idx_v[pl.ds(cbase + i0, 16)]
                for j in range(16):
                    pltpu.async_copy(
                        table_hbm.at[pl.ds(v[j], 1)],
                        rows_v.at[pl.ds(i0 + j, 1)],
                        sem,
                    )

            @pl.loop(0, _CHUNK)
            def _(i):
                pltpu.make_async_copy(
                    table_hbm.at[pl.ds(0, 1)], rows_v.at[pl.ds(0, 1)], sem
                ).wait()

            pltpu.sync_copy(rows_v, out_hbm.at[pl.ds(wbase + cbase, _CHUNK)])

    return gather_kernel(table, idx)


def _loss_body(inv_batch, u_ref, p_ref, n_ref, o_ref):
    i = pl.program_id(0)
    u = u_ref[...]
    p = p_ref[...]
    n = n_ref[...]
    diff = jnp.sum(u * (p - n), axis=1)
    loss_terms = -jnp.log(jax.nn.sigmoid(diff))
    l2 = jnp.sum(u * u + p * p + n * n, axis=1)
    part = jnp.sum(loss_terms + _REG_WEIGHT * l2) * inv_batch

    @pl.when(i == 0)
    def _():
        o_ref[...] = jnp.zeros((1, 1), jnp.float32)

    o_ref[...] += jnp.reshape(part, (1, 1))


def kernel(user_embeddings, item_embeddings, pos_items, neg_items):
    batch, d = user_embeddings.shape
    idx = jnp.concatenate(
        [pos_items.astype(jnp.int32), neg_items.astype(jnp.int32)]
    )
    rows = _sc_gather(item_embeddings, idx)

    blk = 2048
    grid = batch // blk
    out = pl.pallas_call(
        functools.partial(_loss_body, 1.0 / batch),
        grid=(grid,),
        in_specs=[
            pl.BlockSpec((blk, d), lambda i: (i, 0)),
            pl.BlockSpec((blk, d), lambda i: (i, 0)),
            pl.BlockSpec((blk, d), lambda i, g=grid: (i + g, 0)),
        ],
        out_specs=pl.BlockSpec((1, 1), lambda i: (0, 0)),
        out_shape=jax.ShapeDtypeStruct((1, 1), jnp.float32),
    )(user_embeddings, rows, rows)
    return out[0, 0]
